# Initial kernel scaffold; baseline (speedup 1.0000x reference)
import functools

import jax
import jax.numpy as jnp
from jax import lax
from jax.experimental import pallas as pl
from jax.experimental.pallas import tpu as pltpu
from jax.experimental.pallas import tpu_sc as plsc

N = 10000
D = 128
NC = 2
NS = 16
NW = NC * NS
E_PAD = 327680
E_TILE = E_PAD // NW
GROUP = 128
BATCH = 512
GPB = BATCH // GROUP
NBATCH = E_TILE // BATCH
ROWS_SLICE = N // NS


@functools.partial(
    pl.kernel,
    out_type=jax.ShapeDtypeStruct((NC, N, D), jnp.float32),
    mesh=plsc.VectorSubcoreMesh(core_axis_name="c", subcore_axis_name="s"),
    scratch_types=[
        pltpu.VMEM_SHARED((N, D), jnp.float32),
        pltpu.VMEM((BATCH, D), jnp.float32),
        pltpu.VMEM((GPB, GROUP), jnp.int32),
        pltpu.VMEM((GPB, GROUP), jnp.int32),
        pltpu.VMEM((BATCH,), jnp.float32),
    ],
)
def _sc_spmm(cur_hbm, src_hbm, dst_hbm, val_hbm, out_hbm,
             acc, rows, sidx, didx, vals):
    c = lax.axis_index("c")
    s = lax.axis_index("s")
    wid = c * NS + s

    zero = jnp.zeros((16,), jnp.float32)

    @pl.loop(0, BATCH)
    def _(i):
        for j in range(D // 16):
            rows[i, pl.ds(j * 16, 16)] = zero

    base = s * ROWS_SLICE
    pltpu.sync_copy(rows.at[pl.ds(0, BATCH)], acc.at[pl.ds(base, BATCH)])
    pltpu.sync_copy(rows.at[pl.ds(0, ROWS_SLICE - BATCH)],
                    acc.at[pl.ds(base + BATCH, ROWS_SLICE - BATCH)])
    plsc.subcore_barrier()

    e0 = wid * E_TILE
    g0 = wid * (E_TILE // GROUP)

    @pl.loop(0, NBATCH)
    def _(b):
        pltpu.sync_copy(src_hbm.at[pl.ds(g0 + b * GPB, GPB)], sidx)
        pltpu.sync_copy(dst_hbm.at[pl.ds(g0 + b * GPB, GPB)], didx)
        pltpu.sync_copy(val_hbm.at[pl.ds(e0 + b * BATCH, BATCH)], vals)
        for j in range(GPB):
            pltpu.sync_copy(cur_hbm.at[sidx.at[j]],
                            rows.at[pl.ds(j * GROUP, GROUP)])

        @pl.loop(0, BATCH)
        def _(i):
            iv = jnp.full((16,), i, jnp.int32)
            v = plsc.load_gather(vals, [iv])
            for j in range(D // 16):
                sl = pl.ds(j * 16, 16)
                rows[i, sl] = rows[i, sl] * v

        for j in range(GPB):
            pltpu.sync_copy(rows.at[pl.ds(j * GROUP, GROUP)],
                            acc.at[didx.at[j]], add=True)

    plsc.subcore_barrier()
    pltpu.sync_copy(acc.at[pl.ds(base, ROWS_SLICE)],
                    out_hbm.at[c, pl.ds(base, ROWS_SLICE)])


def _combine_body(parts_ref, tot_ref, cur_out, tot_out):
    p = parts_ref[0] + parts_ref[1]
    cur_out[...] = p
    tot_out[...] = tot_ref[...] + p


_tc_combine = pl.pallas_call(
    _combine_body,
    out_shape=[jax.ShapeDtypeStruct((N, D), jnp.float32)] * 2,
)


def _final_body(parts_ref, tot_ref, out_ref):
    p = parts_ref[0] + parts_ref[1]
    out_ref[...] = (tot_ref[...] + p) * 0.25


_tc_final = pl.pallas_call(
    _final_body,
    out_shape=jax.ShapeDtypeStruct((N, D), jnp.float32),
)


def kernel(h, edge_index, adj_values):
    src = edge_index[1].astype(jnp.int32)
    dst = edge_index[0].astype(jnp.int32)
    e = src.shape[0]
    pad = E_PAD - e
    src2d = jnp.concatenate([src, jnp.zeros((pad,), jnp.int32)]).reshape(-1, GROUP)
    dst2d = jnp.concatenate([dst, jnp.zeros((pad,), jnp.int32)]).reshape(-1, GROUP)
    val1d = jnp.concatenate(
        [adj_values.astype(jnp.float32), jnp.zeros((pad,), jnp.float32)])

    cur = h
    tot = h
    for layer in range(3):
        parts = _sc_spmm(cur, src2d, dst2d, val1d)
        if layer < 2:
            cur, tot = _tc_combine(parts, tot)
        else:
            out = _tc_final(parts, tot)
    return out


# speedup vs baseline: 2.3474x; 2.3474x over previous
import dataclasses
import functools

import jax
import jax.numpy as jnp
from jax import lax
from jax.experimental import pallas as pl
from jax.experimental.pallas import tpu as pltpu
from jax.experimental.pallas import tpu_sc as plsc

N = 10000
N_PAD = 10240
D = 128
NC = 2
NS = 16
NW = NC * NS
E_PAD = 327680
E_TILE = E_PAD // NW
GROUP = 128
BATCH = 256
GPB = BATCH // GROUP
NBATCH = E_TILE // BATCH
ROWS_SLICE = N_PAD // NS

_sc_params = pltpu.CompilerParams()
if "needs_layout_passes" in pltpu.CompilerParams.__dataclass_fields__:
    _sc_params = dataclasses.replace(_sc_params, needs_layout_passes=False)


@functools.partial(
    pl.kernel,
    compiler_params=_sc_params,
    out_type=jax.ShapeDtypeStruct((NC, N_PAD, D), jnp.float32),
    mesh=plsc.VectorSubcoreMesh(core_axis_name="c", subcore_axis_name="s"),
    scratch_types=[
        pltpu.VMEM_SHARED((N_PAD, D), jnp.float32),
        pltpu.VMEM((BATCH, D), jnp.float32),
        pltpu.VMEM((GPB, GROUP), jnp.int32),
        pltpu.VMEM((GPB, GROUP), jnp.int32),
        pltpu.VMEM((BATCH,), jnp.float32),
    ],
)
def _sc_spmm(cur_hbm, src_hbm, dst_hbm, val_hbm, out_hbm,
             acc, rows, sidx, didx, vals):
    c = lax.axis_index("c")
    s = lax.axis_index("s")
    wid = c * NS + s

    zero = jnp.zeros((16,), jnp.float32)

    @pl.loop(0, BATCH)
    def _(i):
        for j in range(D // 16):
            rows[i, pl.ds(j * 16, 16)] = zero

    base = s * ROWS_SLICE
    for off in range(0, ROWS_SLICE, BATCH):
        nrow = min(BATCH, ROWS_SLICE - off)
        pltpu.sync_copy(rows.at[pl.ds(0, nrow)],
                        acc.at[pl.ds(base + off, nrow)])
    plsc.subcore_barrier()

    e0 = wid * E_TILE
    g0 = wid * (E_TILE // GROUP)

    @pl.loop(0, NBATCH)
    def _(b):
        pltpu.sync_copy(src_hbm.at[pl.ds(g0 + b * GPB, GPB)], sidx)
        pltpu.sync_copy(dst_hbm.at[pl.ds(g0 + b * GPB, GPB)], didx)
        pltpu.sync_copy(val_hbm.at[pl.ds(e0 + b * BATCH, BATCH)], vals)
        for j in range(GPB):
            pltpu.sync_copy(cur_hbm.at[sidx.at[j]],
                            rows.at[pl.ds(j * GROUP, GROUP)])

        @pl.loop(0, BATCH)
        def _(i):
            iv = jnp.full((16,), i, jnp.int32)
            v = plsc.load_gather(vals, [iv])
            for j in range(D // 16):
                sl = pl.ds(j * 16, 16)
                rows[i, sl] = rows[i, sl] * v

        for j in range(GPB):
            pltpu.sync_copy(rows.at[pl.ds(j * GROUP, GROUP)],
                            acc.at[didx.at[j]], add=True)

    plsc.subcore_barrier()
    pltpu.sync_copy(acc.at[pl.ds(base, ROWS_SLICE)],
                    out_hbm.at[c, pl.ds(base, ROWS_SLICE)])


def _combine_body(parts_ref, tot_ref, cur_out, tot_out):
    p = parts_ref[0] + parts_ref[1]
    cur_out[...] = p
    tot_out[...] = tot_ref[...] + p


_tc_combine = pl.pallas_call(
    _combine_body,
    out_shape=[jax.ShapeDtypeStruct((N_PAD, D), jnp.float32)] * 2,
)


def _final_body(parts_ref, tot_ref, out_ref):
    p = parts_ref[0] + parts_ref[1]
    out_ref[...] = (tot_ref[...] + p) * 0.25


_tc_final = pl.pallas_call(
    _final_body,
    out_shape=jax.ShapeDtypeStruct((N_PAD, D), jnp.float32),
)


def kernel(h, edge_index, adj_values):
    src = edge_index[1].astype(jnp.int32)
    dst = edge_index[0].astype(jnp.int32)
    e = src.shape[0]
    pad = E_PAD - e
    src2d = jnp.concatenate([src, jnp.zeros((pad,), jnp.int32)]).reshape(-1, GROUP)
    dst2d = jnp.concatenate([dst, jnp.zeros((pad,), jnp.int32)]).reshape(-1, GROUP)
    val1d = jnp.concatenate(
        [adj_values.astype(jnp.float32), jnp.zeros((pad,), jnp.float32)])

    hp = jnp.pad(h, ((0, N_PAD - h.shape[0]), (0, 0)))
    cur = hp
    tot = hp
    for layer in range(3):
        parts = _sc_spmm(cur, src2d, dst2d, val1d)
        if layer < 2:
            cur, tot = _tc_combine(parts, tot)
        else:
            out = _tc_final(parts, tot)
    return out[:h.shape[0]]


# speedup vs baseline: 3.3288x; 1.4181x over previous
import dataclasses
import functools

import jax
import jax.numpy as jnp
from jax import lax
from jax.experimental import pallas as pl
from jax.experimental.pallas import tpu as pltpu
from jax.experimental.pallas import tpu_sc as plsc

N = 10000
N_PAD = 10240
D = 128
NC = 2
NS = 16
NW = NC * NS
E_PAD = 327680
E_TILE = E_PAD // NW
GROUP = 128
NB = E_TILE // GROUP
ROWS_SLICE = N_PAD // NS

_sc_params = pltpu.CompilerParams()
if "needs_layout_passes" in pltpu.CompilerParams.__dataclass_fields__:
    _sc_params = dataclasses.replace(_sc_params, needs_layout_passes=False)


@functools.partial(
    pl.kernel,
    compiler_params=_sc_params,
    out_type=jax.ShapeDtypeStruct((NC, N_PAD, D), jnp.float32),
    mesh=plsc.VectorSubcoreMesh(core_axis_name="c", subcore_axis_name="s"),
    scratch_types=(
        [pltpu.VMEM_SHARED((N_PAD, D), jnp.float32)]
        + [pltpu.VMEM((GROUP, D), jnp.float32)] * 2
        + [pltpu.VMEM((1, GROUP), jnp.int32)] * 4
        + [pltpu.VMEM((1, GROUP), jnp.int32)] * 4
        + [pltpu.VMEM((GROUP,), jnp.float32)] * 4
        + [pltpu.SemaphoreType.DMA] * 6
    ),
)
def _sc_spmm(cur_hbm, src_hbm, dst_hbm, val_hbm, out_hbm,
             acc, rows0, rows1, si0, si1, si2, si3, di0, di1, di2, di3,
             va0, va1, va2, va3, sg0, sg1, sj0, sj1, sj2, sj3):
    c = lax.axis_index("c")
    s = lax.axis_index("s")
    wid = c * NS + s
    base = s * ROWS_SLICE
    e0 = wid * E_TILE
    g0 = wid * NB

    rows = (rows0, rows1)
    sidx = (si0, si1, si2, si3)
    didx = (di0, di1, di2, di3)
    vals = (va0, va1, va2, va3)
    sg = (sg0, sg1)
    sj = (sj0, sj1, sj2, sj3)

    def idx_copies(b, k):
        return (
            pltpu.make_async_copy(src_hbm.at[pl.ds(g0 + b, 1)], sidx[k], sj[k]),
            pltpu.make_async_copy(dst_hbm.at[pl.ds(g0 + b, 1)], didx[k], sj[k]),
            pltpu.make_async_copy(val_hbm.at[pl.ds(e0 + b * GROUP, GROUP)],
                                  vals[k], sj[k]),
        )

    def start_idx(b, k):
        for cp in idx_copies(b, k):
            cp.start()

    def wait_idx(b, k):
        for cp in idx_copies(b, k):
            cp.wait()

    def gather_copy(p, k):
        return pltpu.make_async_copy(cur_hbm.at[sidx[k].at[0]], rows[p], sg[p])

    zero = jnp.zeros((16,), jnp.float32)

    @pl.loop(0, GROUP)
    def _(i):
        for j in range(D // 16):
            rows0[i, pl.ds(j * 16, 16)] = zero

    zcopies = [
        pltpu.make_async_copy(rows0, acc.at[pl.ds(base + t * GROUP, GROUP)], sg0)
        for t in range(ROWS_SLICE // GROUP)
    ]
    for cp in zcopies:
        cp.start()
    for k in range(4):
        start_idx(k, k)
    for cp in zcopies:
        cp.wait()
    plsc.subcore_barrier()

    def scale(p, k):
        rb = rows[p]
        vb = vals[k]

        @pl.loop(0, GROUP)
        def _(i):
            iv = jnp.full((16,), i, jnp.int32)
            v = plsc.load_gather(vb, [iv])
            for j in range(D // 16):
                sl = pl.ds(j * 16, 16)
                rb[i, sl] = rb[i, sl] * v

    def step(b, k, p, next_gather, next_idx):
        gather_copy(p, k).wait()
        scale(p, k)
        pltpu.sync_copy(rows[p], acc.at[didx[k].at[0]], add=True)
        if next_idx:
            start_idx(b + 4, k)
        if next_gather:
            kn = (k + 2) % 4
            wait_idx(b + 2, kn)
            gather_copy(p, kn).start()

    wait_idx(0, 0)
    gather_copy(0, 0).start()
    wait_idx(1, 1)
    gather_copy(1, 1).start()

    @pl.loop(0, (NB - 4) // 4)
    def _(t):
        b = t * 4
        step(b + 0, 0, 0, True, True)
        step(b + 1, 1, 1, True, True)
        step(b + 2, 2, 0, True, True)
        step(b + 3, 3, 1, True, True)

    step(NB - 4, 0, 0, True, False)
    step(NB - 3, 1, 1, True, False)
    step(NB - 2, 2, 0, False, False)
    step(NB - 1, 3, 1, False, False)

    plsc.subcore_barrier()
    pltpu.sync_copy(acc.at[pl.ds(base, ROWS_SLICE)],
                    out_hbm.at[c, pl.ds(base, ROWS_SLICE)])


def _combine_body(parts_ref, tot_ref, cur_out, tot_out):
    p = parts_ref[0] + parts_ref[1]
    cur_out[...] = p
    tot_out[...] = tot_ref[...] + p


_tc_combine = pl.pallas_call(
    _combine_body,
    out_shape=[jax.ShapeDtypeStruct((N_PAD, D), jnp.float32)] * 2,
)


def _final_body(parts_ref, tot_ref, out_ref):
    p = parts_ref[0] + parts_ref[1]
    out_ref[...] = (tot_ref[...] + p) * 0.25


_tc_final = pl.pallas_call(
    _final_body,
    out_shape=jax.ShapeDtypeStruct((N_PAD, D), jnp.float32),
)


def kernel(h, edge_index, adj_values):
    src = edge_index[1].astype(jnp.int32)
    dst = edge_index[0].astype(jnp.int32)
    e = src.shape[0]
    pad = E_PAD - e
    src2d = jnp.concatenate([src, jnp.zeros((pad,), jnp.int32)]).reshape(-1, GROUP)
    dst2d = jnp.concatenate([dst, jnp.zeros((pad,), jnp.int32)]).reshape(-1, GROUP)
    val1d = jnp.concatenate(
        [adj_values.astype(jnp.float32), jnp.zeros((pad,), jnp.float32)])

    hp = jnp.pad(h, ((0, N_PAD - h.shape[0]), (0, 0)))
    cur = hp
    tot = hp
    for layer in range(3):
        parts = _sc_spmm(cur, src2d, dst2d, val1d)
        if layer < 2:
            cur, tot = _tc_combine(parts, tot)
        else:
            out = _tc_final(parts, tot)
    return out[:h.shape[0]]


# speedup vs baseline: 3.5670x; 1.0715x over previous
import dataclasses
import functools

import jax
import jax.numpy as jnp
from jax import lax
from jax.experimental import pallas as pl
from jax.experimental.pallas import tpu as pltpu
from jax.experimental.pallas import tpu_sc as plsc

N = 10000
N_PAD = 10240
D = 128
NC = 2
NS = 16
NW = NC * NS
E_PAD = 327680
E_TILE = E_PAD // NW
GROUP = 128
NB0 = 120
NB1 = 40
E_C0 = NB0 * GROUP * NS
ROWS_SLICE = N_PAD // NS

_sc_params = pltpu.CompilerParams()
if "needs_layout_passes" in pltpu.CompilerParams.__dataclass_fields__:
    _sc_params = dataclasses.replace(_sc_params, needs_layout_passes=False)


@functools.partial(
    pl.kernel,
    compiler_params=_sc_params,
    out_type=jax.ShapeDtypeStruct((NC, N_PAD, D), jnp.float32),
    mesh=plsc.VectorSubcoreMesh(core_axis_name="c", subcore_axis_name="s"),
    scratch_types=(
        [pltpu.VMEM_SHARED((N_PAD, D), jnp.float32)]
        + [pltpu.VMEM((GROUP, D), jnp.float32)] * 2
        + [pltpu.VMEM((1, GROUP), jnp.int32)] * 4
        + [pltpu.VMEM((1, GROUP), jnp.int32)] * 4
        + [pltpu.VMEM((GROUP,), jnp.float32)] * 4
        + [pltpu.SemaphoreType.DMA] * 6
    ),
)
def _sc_spmm(cur_hbm, src_hbm, dst_hbm, val_hbm, out_hbm,
             acc, rows0, rows1, si0, si1, si2, si3, di0, di1, di2, di3,
             va0, va1, va2, va3, sg0, sg1, sj0, sj1, sj2, sj3):
    c = lax.axis_index("c")
    s = lax.axis_index("s")
    base = s * ROWS_SLICE
    nb = jnp.where(c == 0, NB0, NB1)
    e0 = jnp.where(c == 0, s * (NB0 * GROUP), E_C0 + s * (NB1 * GROUP))
    g0 = e0 // GROUP

    rows = (rows0, rows1)
    sidx = (si0, si1, si2, si3)
    didx = (di0, di1, di2, di3)
    vals = (va0, va1, va2, va3)
    sg = (sg0, sg1)
    sj = (sj0, sj1, sj2, sj3)

    def idx_copies(b, k):
        return (
            pltpu.make_async_copy(src_hbm.at[pl.ds(g0 + b, 1)], sidx[k], sj[k]),
            pltpu.make_async_copy(dst_hbm.at[pl.ds(g0 + b, 1)], didx[k], sj[k]),
            pltpu.make_async_copy(val_hbm.at[pl.ds(e0 + b * GROUP, GROUP)],
                                  vals[k], sj[k]),
        )

    def start_idx(b, k):
        for cp in idx_copies(b, k):
            cp.start()

    def wait_idx(b, k):
        for cp in idx_copies(b, k):
            cp.wait()

    def gather_copy(p, k):
        return pltpu.make_async_copy(cur_hbm.at[sidx[k].at[0]], rows[p], sg[p])

    zero = jnp.zeros((16,), jnp.float32)

    @pl.loop(0, GROUP)
    def _(i):
        for j in range(D // 16):
            rows0[i, pl.ds(j * 16, 16)] = zero

    zcopies = [
        pltpu.make_async_copy(rows0, acc.at[pl.ds(base + t * GROUP, GROUP)], sg0)
        for t in range(ROWS_SLICE // GROUP)
    ]
    for cp in zcopies:
        cp.start()
    for k in range(4):
        start_idx(k, k)
    for cp in zcopies:
        cp.wait()
    plsc.subcore_barrier()

    def scale(p, k):
        rb = rows[p]
        vb = vals[k]

        @pl.loop(0, GROUP)
        def _(i):
            iv = jnp.full((16,), i, jnp.int32)
            v = plsc.load_gather(vb, [iv])
            for j in range(D // 16):
                sl = pl.ds(j * 16, 16)
                rb[i, sl] = rb[i, sl] * v

    def step(b, k, p, next_gather, next_idx):
        gather_copy(p, k).wait()
        scale(p, k)
        pltpu.sync_copy(rows[p], acc.at[didx[k].at[0]], add=True)
        if next_idx:
            start_idx(b + 4, k)
        if next_gather:
            kn = (k + 2) % 4
            wait_idx(b + 2, kn)
            gather_copy(p, kn).start()

    wait_idx(0, 0)
    gather_copy(0, 0).start()
    wait_idx(1, 1)
    gather_copy(1, 1).start()

    @pl.loop(0, (nb - 4) // 4)
    def _(t):
        b = t * 4
        step(b + 0, 0, 0, True, True)
        step(b + 1, 1, 1, True, True)
        step(b + 2, 2, 0, True, True)
        step(b + 3, 3, 1, True, True)

    step(nb - 4, 0, 0, True, False)
    step(nb - 3, 1, 1, True, False)
    step(nb - 2, 2, 0, False, False)
    step(nb - 1, 3, 1, False, False)

    plsc.subcore_barrier()
    pltpu.sync_copy(acc.at[pl.ds(base, ROWS_SLICE)],
                    out_hbm.at[c, pl.ds(base, ROWS_SLICE)])


def _combine_body(parts_ref, tot_ref, cur_out, tot_out):
    p = parts_ref[0] + parts_ref[1]
    cur_out[...] = p
    tot_out[...] = tot_ref[...] + p


_tc_combine = pl.pallas_call(
    _combine_body,
    out_shape=[jax.ShapeDtypeStruct((N_PAD, D), jnp.float32)] * 2,
)


def _final_body(parts_ref, tot_ref, out_ref):
    p = parts_ref[0] + parts_ref[1]
    out_ref[...] = (tot_ref[...] + p) * 0.25


_tc_final = pl.pallas_call(
    _final_body,
    out_shape=jax.ShapeDtypeStruct((N_PAD, D), jnp.float32),
)


def kernel(h, edge_index, adj_values):
    src = edge_index[1].astype(jnp.int32)
    dst = edge_index[0].astype(jnp.int32)
    e = src.shape[0]
    pad = E_PAD - e
    src2d = jnp.concatenate([src, jnp.zeros((pad,), jnp.int32)]).reshape(-1, GROUP)
    dst2d = jnp.concatenate([dst, jnp.zeros((pad,), jnp.int32)]).reshape(-1, GROUP)
    val1d = jnp.concatenate(
        [adj_values.astype(jnp.float32), jnp.zeros((pad,), jnp.float32)])

    hp = jnp.pad(h, ((0, N_PAD - h.shape[0]), (0, 0)))
    cur = hp
    tot = hp
    for layer in range(3):
        parts = _sc_spmm(cur, src2d, dst2d, val1d)
        if layer < 2:
            cur, tot = _tc_combine(parts, tot)
        else:
            out = _tc_final(parts, tot)
    return out[:h.shape[0]]
